# baseline (speedup 1.0000x reference)
import functools

import jax
import jax.numpy as jnp
from jax.experimental import pallas as pl
from jax.experimental.pallas import tpu as pltpu

_ALPHA = 0.1
_RATE = 0.7

_TM1 = 512
_TM2 = 512
_VMEM = 56 * 1024 * 1024


def _leaky(v):
    return jnp.where(v > 0.0, v, _ALPHA * v)


def _dot(a, b):
    return jnp.dot(a, b, preferred_element_type=jnp.float32)


def _dot_nt(a, b):
    return jax.lax.dot_general(
        a, b, (((1,), (1,)), ((), ())), preferred_element_type=jnp.float32)


def _layer1_body(adj_ref, x_ref, w1_ref, b1_ref, w3m_ref, w3l_ref, o_ref):
    sup1 = _dot(x_ref[...], w1_ref[...])
    h = _leaky(_dot(adj_ref[...], sup1) + b1_ref[...])
    o_ref[...] = jnp.concatenate([_dot(h, w3m_ref[...]), _dot(h, w3l_ref[...])],
                                 axis=1)


def _layer1(adj, x, w1, b1, w3m, w3l):
    n_rows, n_k = adj.shape
    n_in = x.shape[1]
    n_hid = w1.shape[1]
    n_out = w3m.shape[1] + w3l.shape[1]
    tm = min(_TM1, n_rows)
    return pl.pallas_call(
        _layer1_body,
        grid=(n_rows // tm,),
        in_specs=[
            pl.BlockSpec((tm, n_k), lambda i: (i, 0)),
            pl.BlockSpec((n_k, n_in), lambda i: (0, 0)),
            pl.BlockSpec((n_in, n_hid), lambda i: (0, 0)),
            pl.BlockSpec((1, n_hid), lambda i: (0, 0)),
            pl.BlockSpec((n_hid, w3m.shape[1]), lambda i: (0, 0)),
            pl.BlockSpec((n_hid, w3l.shape[1]), lambda i: (0, 0)),
        ],
        out_specs=pl.BlockSpec((tm, n_out), lambda i: (i, 0)),
        out_shape=jax.ShapeDtypeStruct((n_rows, n_out), jnp.float32),
        compiler_params=pltpu.CompilerParams(
            dimension_semantics=("parallel",),
            vmem_limit_bytes=_VMEM,
        ),
    )(adj, x, w1, b1.reshape(1, -1), w3m, w3l)


def _layer2_union_body(adj_s_ref, adj_t_ref, sup_s_ref, sup_t_ref,
                       b3m_ref, b3l_ref, b4m_ref, b4l_ref,
                       sf_ref, tf_ref,
                       wsm_ref, wsl_ref, wtm_ref, wtl_ref,
                       bsm_ref, bsl_ref, btm_ref, btl_ref,
                       om_ref, ol_ref, *, fdim):
    rate = jnp.float32(_RATE)
    b3 = jnp.concatenate([b3m_ref[...], b3l_ref[...]], axis=1)
    b4 = jnp.concatenate([b4m_ref[...], b4l_ref[...]], axis=1)
    s_cat = _leaky(_dot(adj_s_ref[...], sup_s_ref[...]) + b3)
    t_cat = _leaky(_dot(adj_t_ref[...], sup_t_ref[...]) + b4)
    sf = sf_ref[...]
    tf = tf_ref[...]
    wsm, wsl = wsm_ref[...], wsl_ref[...]
    wtm, wtl = wtm_ref[...], wtl_ref[...]
    mean = (rate * (_dot_nt(s_cat[:, :fdim], wsm[:, :fdim])
                    + _dot_nt(sf, wsm[:, fdim:]) + bsm_ref[...])
            + (1.0 - rate) * (_dot_nt(t_cat[:, :fdim], wtm[:, :fdim])
                              + _dot_nt(tf, wtm[:, fdim:]) + btm_ref[...]))
    logstd = (rate * (_dot_nt(s_cat[:, fdim:], wsl[:, :fdim])
                      + _dot_nt(sf, wsl[:, fdim:]) + bsl_ref[...])
              + (1.0 - rate) * (_dot_nt(t_cat[:, fdim:], wtl[:, :fdim])
                                + _dot_nt(tf, wtl[:, fdim:]) + btl_ref[...]))
    om_ref[...] = mean
    ol_ref[...] = logstd


def kernel(gc1_w, gc1_b, gc2_w, gc2_b,
           gc3_mean_w, gc3_mean_b, gc3_logstd_w, gc3_logstd_b,
           gc4_mean_w, gc4_mean_b, gc4_logstd_w, gc4_logstd_b,
           union_source_mean_w, union_source_mean_b,
           union_source_logstd_w, union_source_logstd_b,
           union_target_mean_w, union_target_mean_b,
           union_target_logstd_w, union_target_logstd_b,
           source_ufea, target_ufea,
           source_UV_adj, source_VU_adj, target_UV_adj, target_VU_adj):
    fdim = source_ufea.shape[1]
    n_user = source_ufea.shape[0]
    two_f = 2 * fdim

    sup_s = _layer1(source_VU_adj, source_ufea, gc1_w, gc1_b,
                    gc3_mean_w, gc3_logstd_w)
    sup_t = _layer1(target_VU_adj, target_ufea, gc2_w, gc2_b,
                    gc4_mean_w, gc4_logstd_w)

    n_item_s = source_UV_adj.shape[1]
    n_item_t = target_UV_adj.shape[1]
    tm2 = min(_TM2, n_user)

    row_spec = lambda i: (i, 0)
    pin = lambda i: (0, 0)

    mean, logstd = pl.pallas_call(
        functools.partial(_layer2_union_body, fdim=fdim),
        grid=(n_user // tm2,),
        in_specs=[
            pl.BlockSpec((tm2, n_item_s), row_spec),
            pl.BlockSpec((tm2, n_item_t), row_spec),
            pl.BlockSpec((n_item_s, two_f), pin),
            pl.BlockSpec((n_item_t, two_f), pin),
            pl.BlockSpec((1, fdim), pin),
            pl.BlockSpec((1, fdim), pin),
            pl.BlockSpec((1, fdim), pin),
            pl.BlockSpec((1, fdim), pin),
            pl.BlockSpec((tm2, fdim), row_spec),
            pl.BlockSpec((tm2, fdim), row_spec),
            pl.BlockSpec((fdim, two_f), pin),
            pl.BlockSpec((fdim, two_f), pin),
            pl.BlockSpec((fdim, two_f), pin),
            pl.BlockSpec((fdim, two_f), pin),
            pl.BlockSpec((1, fdim), pin),
            pl.BlockSpec((1, fdim), pin),
            pl.BlockSpec((1, fdim), pin),
            pl.BlockSpec((1, fdim), pin),
        ],
        out_specs=[
            pl.BlockSpec((tm2, fdim), row_spec),
            pl.BlockSpec((tm2, fdim), row_spec),
        ],
        out_shape=[
            jax.ShapeDtypeStruct((n_user, fdim), jnp.float32),
            jax.ShapeDtypeStruct((n_user, fdim), jnp.float32),
        ],
        compiler_params=pltpu.CompilerParams(
            dimension_semantics=("parallel",),
            vmem_limit_bytes=_VMEM,
        ),
    )(source_UV_adj, target_UV_adj, sup_s, sup_t,
      gc3_mean_b.reshape(1, -1), gc3_logstd_b.reshape(1, -1),
      gc4_mean_b.reshape(1, -1), gc4_logstd_b.reshape(1, -1),
      source_ufea, target_ufea,
      union_source_mean_w, union_source_logstd_w,
      union_target_mean_w, union_target_logstd_w,
      union_source_mean_b.reshape(1, -1), union_source_logstd_b.reshape(1, -1),
      union_target_mean_b.reshape(1, -1), union_target_logstd_b.reshape(1, -1))

    return mean, logstd
